# baseline (speedup 1.0000x reference)
import functools

import jax
import jax.numpy as jnp
from jax.experimental import pallas as pl
from jax.experimental.pallas import tpu as pltpu

N_ROWS = 100000
D_NODE = 140
GRP = 20
ROWS = N_ROWS // GRP
COLS = D_NODE * GRP
BLK = 200
NBLK = ROWS // BLK
T_STEPS = 200


def _fused_kernel(x_ref, w_ref, bg_ref, cfg_ref, wih0_ref, whh0_ref, b0_ref,
                  wih1_ref, whh1_ref, b1_ref, wfc_ref, bfc_ref,
                  out_ref, acc_ref, g0_ref):
    i = pl.program_id(0)
    x = x_ref[...]
    y = jnp.dot(x, w_ref[...], preferred_element_type=jnp.float32)
    yg = jnp.maximum(y[:, :GRP] + bg_ref[0, 0], 0.0)
    part = jnp.sum(yg) + jnp.sum(y[:, GRP])

    @pl.when(i == 0)
    def _init():
        acc_ref[0, 0] = part

    @pl.when(i > 0)
    def _accum():
        acc_ref[0, 0] = acc_ref[0, 0] + part

    @pl.when(i == NBLK - 1)
    def _finish():
        g0_ref[...] = jnp.dot(cfg_ref[...], wih0_ref[...],
                              preferred_element_type=jnp.float32)
        whh0 = whh0_ref[...]
        b0 = b0_ref[...]
        wih1 = wih1_ref[...]
        whh1 = whh1_ref[...]
        b1 = b1_ref[...]

        def step(t, carry):
            h0, c0, h1, c1 = carry
            gates0 = g0_ref[pl.ds(t, 1), :] + h0 * whh0 + b0
            s0 = jax.nn.sigmoid(gates0)
            t0 = jnp.tanh(gates0)
            c0n = s0[0, 1] * c0 + s0[0, 0] * t0[0, 2]
            h0n = s0[0, 3] * jnp.tanh(c0n)
            gates1 = h0n * wih1 + h1 * whh1 + b1
            s1 = jax.nn.sigmoid(gates1)
            t1 = jnp.tanh(gates1)
            c1n = s1[0, 1] * c1 + s1[0, 0] * t1[0, 2]
            h1n = s1[0, 3] * jnp.tanh(c1n)
            return (h0n, c0n, h1n, c1n)

        z = jnp.float32(0.0)
        h0, c0, h1, c1 = jax.lax.fori_loop(0, T_STEPS, step, (z, z, z, z))

        gnn = acc_ref[0, 0] * jnp.float32(1.0 / N_ROWS)
        wfc = wfc_ref[...]
        res = gnn * wfc[0, 0] + h1 * wfc[0, 1] + bfc_ref[0, 0]
        out_ref[...] = jnp.reshape(res, (1, 1))


@functools.partial(jax.jit, static_argnames=())
def _run(node_feat, cfg, w_col, bg, wih0_t, whh0r, b0r, wih1r, whh1r, b1r,
         wfc, bfc):
    full = lambda shape: pl.BlockSpec(shape, lambda i: (0, 0))
    return pl.pallas_call(
        _fused_kernel,
        grid=(NBLK,),
        in_specs=[
            pl.BlockSpec((BLK, COLS), lambda i: (i, 0)),
            full((COLS, GRP + 1)),
            full((1, 1)),
            full((T_STEPS, cfg.shape[1])),
            full(wih0_t.shape),
            full((1, 4)),
            full((1, 4)),
            full((1, 4)),
            full((1, 4)),
            full((1, 4)),
            full((1, 2)),
            full((1, 1)),
        ],
        out_specs=pl.BlockSpec((1, 1), lambda i: (0, 0)),
        out_shape=jax.ShapeDtypeStruct((1, 1), jnp.float32),
        scratch_shapes=[
            pltpu.SMEM((1, 1), jnp.float32),
            pltpu.VMEM((T_STEPS, 4), jnp.float32),
        ],
    )(node_feat, w_col, bg, cfg, wih0_t, whh0r, b0r, wih1r, whh1r, b1r,
      wfc, bfc)


def kernel(node_feat, edge_index, config_feat, W_gnn, b_gnn, Wih0, Whh0,
           bih0, bhh0, Wih1, Whh1, bih1, bhh1, W_fc, b_fc):
    cfg = config_feat.reshape(config_feat.shape[1], config_feat.shape[2])
    xr = node_feat.reshape(ROWS, COLS)
    wrow = W_gnn.reshape(D_NODE, 1)
    wblk = jnp.kron(jnp.eye(GRP, dtype=jnp.float32), wrow)
    sel = jnp.zeros((D_NODE, 1), jnp.float32).at[0, 0].set(1.0)
    sel = jnp.tile(sel, (GRP, 1))
    w_col = jnp.concatenate([wblk, sel], axis=1)
    bg = b_gnn.reshape(1, 1)
    wih0_t = Wih0.T
    whh0r = Whh0.T.reshape(1, 4)
    b0r = (bih0 + bhh0).reshape(1, 4)
    wih1r = Wih1.T.reshape(1, 4)
    whh1r = Whh1.T.reshape(1, 4)
    b1r = (bih1 + bhh1).reshape(1, 4)
    wfc = W_fc.reshape(1, 2)
    bfc = b_fc.reshape(1, 1)
    return _run(xr, cfg, w_col, bg, wih0_t, whh0r, b0r, wih1r, whh1r,
                b1r, wfc, bfc)


# speedup vs baseline: 3.3628x; 3.3628x over previous
import functools

import jax
import jax.numpy as jnp
from jax.experimental import pallas as pl
from jax.experimental.pallas import tpu as pltpu

N_ROWS = 100000
D_NODE = 140
BLK = 10000
NBLK = N_ROWS // BLK
T_STEPS = 200


def _fused_kernel(x_ref, w_ref, bg_ref, cfg_ref, wih0_ref, whh0_ref, b0_ref,
                  wih1_ref, whh1_ref, b1_ref, wfc_ref, bfc_ref,
                  out_ref, acc_ref, g0_ref):
    i = pl.program_id(0)
    x = x_ref[...]
    y = jnp.dot(x, w_ref[...], preferred_element_type=jnp.float32)
    part = (jnp.sum(jnp.maximum(y[:, 0] + bg_ref[0, 0], 0.0))
            + jnp.sum(y[:, 1]))

    @pl.when(i == 0)
    def _init():
        acc_ref[0, 0] = part

    @pl.when(i > 0)
    def _accum():
        acc_ref[0, 0] = acc_ref[0, 0] + part

    @pl.when(i == NBLK - 1)
    def _finish():
        g0_ref[...] = jnp.dot(cfg_ref[...], wih0_ref[...],
                              preferred_element_type=jnp.float32)
        whh0 = whh0_ref[...]
        b0 = b0_ref[...]
        wih1 = wih1_ref[...]
        whh1 = whh1_ref[...]
        b1 = b1_ref[...]

        def step(t, carry):
            h0, c0, h1, c1 = carry
            gates0 = g0_ref[pl.ds(t, 1), :] + h0 * whh0 + b0
            s0 = jax.nn.sigmoid(gates0)
            t0 = jnp.tanh(gates0)
            c0n = s0[0, 1] * c0 + s0[0, 0] * t0[0, 2]
            h0n = s0[0, 3] * jnp.tanh(c0n)
            gates1 = h0n * wih1 + h1 * whh1 + b1
            s1 = jax.nn.sigmoid(gates1)
            t1 = jnp.tanh(gates1)
            c1n = s1[0, 1] * c1 + s1[0, 0] * t1[0, 2]
            h1n = s1[0, 3] * jnp.tanh(c1n)
            return (h0n, c0n, h1n, c1n)

        z = jnp.float32(0.0)
        h0, c0, h1, c1 = jax.lax.fori_loop(0, T_STEPS, step, (z, z, z, z))

        gnn = acc_ref[0, 0] * jnp.float32(1.0 / N_ROWS)
        wfc = wfc_ref[...]
        res = gnn * wfc[0, 0] + h1 * wfc[0, 1] + bfc_ref[0, 0]
        out_ref[...] = jnp.reshape(res, (1, 1))


@functools.partial(jax.jit, static_argnames=())
def _run(node_feat, cfg, w_col, bg, wih0_t, whh0r, b0r, wih1r, whh1r, b1r,
         wfc, bfc):
    full = lambda shape: pl.BlockSpec(shape, lambda i: (0, 0))
    return pl.pallas_call(
        _fused_kernel,
        grid=(NBLK,),
        in_specs=[
            pl.BlockSpec((BLK, D_NODE), lambda i: (i, 0)),
            full((D_NODE, 2)),
            full((1, 1)),
            full((T_STEPS, cfg.shape[1])),
            full(wih0_t.shape),
            full((1, 4)),
            full((1, 4)),
            full((1, 4)),
            full((1, 4)),
            full((1, 4)),
            full((1, 2)),
            full((1, 1)),
        ],
        out_specs=pl.BlockSpec((1, 1), lambda i: (0, 0)),
        out_shape=jax.ShapeDtypeStruct((1, 1), jnp.float32),
        scratch_shapes=[
            pltpu.SMEM((1, 1), jnp.float32),
            pltpu.VMEM((T_STEPS, 4), jnp.float32),
        ],
    )(node_feat, w_col, bg, cfg, wih0_t, whh0r, b0r, wih1r, whh1r, b1r,
      wfc, bfc)


def kernel(node_feat, edge_index, config_feat, W_gnn, b_gnn, Wih0, Whh0,
           bih0, bhh0, Wih1, Whh1, bih1, bhh1, W_fc, b_fc):
    cfg = config_feat.reshape(config_feat.shape[1], config_feat.shape[2])
    sel = jnp.zeros((D_NODE, 1), jnp.float32).at[0, 0].set(1.0)
    w_col = jnp.concatenate([W_gnn.reshape(D_NODE, 1), sel], axis=1)
    bg = b_gnn.reshape(1, 1)
    wih0_t = Wih0.T
    whh0r = Whh0.T.reshape(1, 4)
    b0r = (bih0 + bhh0).reshape(1, 4)
    wih1r = Wih1.T.reshape(1, 4)
    whh1r = Whh1.T.reshape(1, 4)
    b1r = (bih1 + bhh1).reshape(1, 4)
    wfc = W_fc.reshape(1, 2)
    bfc = b_fc.reshape(1, 1)
    return _run(node_feat, cfg, w_col, bg, wih0_t, whh0r, b0r, wih1r, whh1r,
                b1r, wfc, bfc)


# speedup vs baseline: 3.3998x; 1.0110x over previous
import functools

import jax
import jax.numpy as jnp
from jax.experimental import pallas as pl
from jax.experimental.pallas import tpu as pltpu

N_ROWS = 100000
D_NODE = 140
BLK = 10000
NBLK = N_ROWS // BLK
T_STEPS = 200
T_CHUNK = T_STEPS // NBLK


def _fused_kernel(x_ref, w_ref, bg_ref, cfg_ref, wih0_ref, whh0_ref, b0_ref,
                  wih1_ref, whh1_ref, b1_ref, wfc_ref, bfc_ref,
                  out_ref, acc_ref, st_ref, g0_ref):
    i = pl.program_id(0)
    x = x_ref[...]
    y = jnp.dot(x, w_ref[...], preferred_element_type=jnp.float32)
    part = (jnp.sum(jnp.maximum(y[:, 0] + bg_ref[0, 0], 0.0))
            + jnp.sum(y[:, 1]))

    @pl.when(i == 0)
    def _init():
        acc_ref[0, 0] = part
        g0_ref[...] = jnp.dot(cfg_ref[...], wih0_ref[...],
                              preferred_element_type=jnp.float32)
        st_ref[0, 0] = 0.0
        st_ref[0, 1] = 0.0
        st_ref[0, 2] = 0.0
        st_ref[0, 3] = 0.0

    @pl.when(i > 0)
    def _accum():
        acc_ref[0, 0] = acc_ref[0, 0] + part

    whh0 = whh0_ref[...]
    b0 = b0_ref[...]
    wih1 = wih1_ref[...]
    whh1 = whh1_ref[...]
    b1 = b1_ref[...]

    def step(t, carry):
        h0, c0, h1, c1 = carry
        gates0 = g0_ref[pl.ds(t, 1), :] + h0 * whh0 + b0
        s0 = jax.nn.sigmoid(gates0)
        t0 = jnp.tanh(gates0)
        c0n = s0[0, 1] * c0 + s0[0, 0] * t0[0, 2]
        h0n = s0[0, 3] * jnp.tanh(c0n)
        gates1 = h0n * wih1 + h1 * whh1 + b1
        s1 = jax.nn.sigmoid(gates1)
        t1 = jnp.tanh(gates1)
        c1n = s1[0, 1] * c1 + s1[0, 0] * t1[0, 2]
        h1n = s1[0, 3] * jnp.tanh(c1n)
        return (h0n, c0n, h1n, c1n)

    t0_base = i * T_CHUNK
    carry0 = (st_ref[0, 0], st_ref[0, 1], st_ref[0, 2], st_ref[0, 3])
    h0, c0, h1, c1 = jax.lax.fori_loop(t0_base, t0_base + T_CHUNK, step,
                                       carry0)
    st_ref[0, 0] = h0
    st_ref[0, 1] = c0
    st_ref[0, 2] = h1
    st_ref[0, 3] = c1

    @pl.when(i == NBLK - 1)
    def _finish():
        gnn = acc_ref[0, 0] * jnp.float32(1.0 / N_ROWS)
        wfc = wfc_ref[...]
        res = gnn * wfc[0, 0] + st_ref[0, 2] * wfc[0, 1] + bfc_ref[0, 0]
        out_ref[...] = jnp.reshape(res, (1, 1))


@jax.jit
def _run(node_feat, cfg, w_col, bg, wih0_t, whh0r, b0r, wih1r, whh1r, b1r,
         wfc, bfc):
    full = lambda shape: pl.BlockSpec(shape, lambda i: (0, 0))
    return pl.pallas_call(
        _fused_kernel,
        grid=(NBLK,),
        in_specs=[
            pl.BlockSpec((BLK, D_NODE), lambda i: (i, 0)),
            full((D_NODE, 2)),
            full((1, 1)),
            full((T_STEPS, cfg.shape[1])),
            full(wih0_t.shape),
            full((1, 4)),
            full((1, 4)),
            full((1, 4)),
            full((1, 4)),
            full((1, 4)),
            full((1, 2)),
            full((1, 1)),
        ],
        out_specs=pl.BlockSpec((1, 1), lambda i: (0, 0)),
        out_shape=jax.ShapeDtypeStruct((1, 1), jnp.float32),
        scratch_shapes=[
            pltpu.SMEM((1, 1), jnp.float32),
            pltpu.SMEM((1, 4), jnp.float32),
            pltpu.VMEM((T_STEPS, 4), jnp.float32),
        ],
    )(node_feat, w_col, bg, cfg, wih0_t, whh0r, b0r, wih1r, whh1r, b1r,
      wfc, bfc)


def kernel(node_feat, edge_index, config_feat, W_gnn, b_gnn, Wih0, Whh0,
           bih0, bhh0, Wih1, Whh1, bih1, bhh1, W_fc, b_fc):
    cfg = config_feat.reshape(config_feat.shape[1], config_feat.shape[2])
    sel = jnp.zeros((D_NODE, 1), jnp.float32).at[0, 0].set(1.0)
    w_col = jnp.concatenate([W_gnn.reshape(D_NODE, 1), sel], axis=1)
    bg = b_gnn.reshape(1, 1)
    wih0_t = Wih0.T
    whh0r = Whh0.T.reshape(1, 4)
    b0r = (bih0 + bhh0).reshape(1, 4)
    wih1r = Wih1.T.reshape(1, 4)
    whh1r = Whh1.T.reshape(1, 4)
    b1r = (bih1 + bhh1).reshape(1, 4)
    wfc = W_fc.reshape(1, 2)
    bfc = b_fc.reshape(1, 1)
    return _run(node_feat, cfg, w_col, bg, wih0_t, whh0r, b0r, wih1r, whh1r,
                b1r, wfc, bfc)


# speedup vs baseline: 3.4126x; 1.0038x over previous
import jax
import jax.numpy as jnp
from jax.experimental import pallas as pl
from jax.experimental.pallas import tpu as pltpu

N_ROWS = 100000
D_NODE = 140
BLK = 10000
NBLK = N_ROWS // BLK
T_STEPS = 200
T_CHUNK = T_STEPS // NBLK


def _stream_kernel(x_ref, w_ref, bg_ref, cfg_ref, wih0_ref, whh0_ref, b0_ref,
                   wih1_ref, whh1_ref, b1_ref, out_ref, st_ref, g0_ref):
    i = pl.program_id(0)
    x = x_ref[...]
    y = jnp.dot(x, w_ref[...], preferred_element_type=jnp.float32)
    part = (jnp.sum(jnp.maximum(y[:, 0] + bg_ref[0, 0], 0.0))
            + jnp.sum(y[:, 1]))

    @pl.when(i == 0)
    def _init():
        g0_ref[...] = jnp.dot(cfg_ref[...], wih0_ref[...],
                              preferred_element_type=jnp.float32)
        st_ref[0, 0] = 0.0
        st_ref[0, 1] = 0.0
        st_ref[0, 2] = 0.0
        st_ref[0, 3] = 0.0

    whh0 = whh0_ref[...]
    b0 = b0_ref[...]
    wih1 = wih1_ref[...]
    whh1 = whh1_ref[...]
    b1 = b1_ref[...]

    def step(t, carry):
        h0, c0, h1, c1 = carry
        gates0 = g0_ref[pl.ds(t, 1), :] + h0 * whh0 + b0
        s0 = jax.nn.sigmoid(gates0)
        t0 = jnp.tanh(gates0)
        c0n = s0[0, 1] * c0 + s0[0, 0] * t0[0, 2]
        h0n = s0[0, 3] * jnp.tanh(c0n)
        gates1 = h0n * wih1 + h1 * whh1 + b1
        s1 = jax.nn.sigmoid(gates1)
        t1 = jnp.tanh(gates1)
        c1n = s1[0, 1] * c1 + s1[0, 0] * t1[0, 2]
        h1n = s1[0, 3] * jnp.tanh(c1n)
        return (h0n, c0n, h1n, c1n)

    t_base = i * T_CHUNK
    carry0 = (st_ref[0, 0], st_ref[0, 1], st_ref[0, 2], st_ref[0, 3])
    h0, c0, h1, c1 = jax.lax.fori_loop(t_base, t_base + T_CHUNK, step, carry0)
    st_ref[0, 0] = h0
    st_ref[0, 1] = c0
    st_ref[0, 2] = h1
    st_ref[0, 3] = c1

    lane = jax.lax.broadcasted_iota(jnp.int32, (1, 1, 128), 2)
    out_ref[...] = jnp.where(lane == 1, h1, jnp.full((1, 1, 128), part))


def _combine_kernel(p_ref, wfc_ref, bfc_ref, out_ref):
    total = jnp.sum(p_ref[:, 0, 0])
    h1 = p_ref[NBLK - 1, 0, 1]
    gnn = total * jnp.float32(1.0 / N_ROWS)
    wfc = wfc_ref[...]
    res = gnn * wfc[0, 0] + h1 * wfc[0, 1] + bfc_ref[0, 0]
    out_ref[...] = jnp.reshape(res, (1, 1))


@jax.jit
def _run(node_feat, cfg, w_col, bg, wih0_t, whh0r, b0r, wih1r, whh1r, b1r,
         wfc, bfc):
    full = lambda shape: pl.BlockSpec(shape, lambda i: (0, 0))
    partials = pl.pallas_call(
        _stream_kernel,
        grid=(NBLK,),
        in_specs=[
            pl.BlockSpec((BLK, D_NODE), lambda i: (i, 0)),
            full((D_NODE, 2)),
            full((1, 1)),
            full((T_STEPS, cfg.shape[1])),
            full(wih0_t.shape),
            full((1, 4)),
            full((1, 4)),
            full((1, 4)),
            full((1, 4)),
            full((1, 4)),
        ],
        out_specs=pl.BlockSpec((1, 1, 128), lambda i: (i, 0, 0)),
        out_shape=jax.ShapeDtypeStruct((NBLK, 1, 128), jnp.float32),
        scratch_shapes=[
            pltpu.SMEM((1, 4), jnp.float32),
            pltpu.VMEM((T_STEPS, 4), jnp.float32),
        ],
        compiler_params=pltpu.CompilerParams(
            dimension_semantics=("arbitrary",)),
    )(node_feat, w_col, bg, cfg, wih0_t, whh0r, b0r, wih1r, whh1r, b1r)

    return pl.pallas_call(
        _combine_kernel,
        out_shape=jax.ShapeDtypeStruct((1, 1), jnp.float32),
    )(partials, wfc, bfc)


def kernel(node_feat, edge_index, config_feat, W_gnn, b_gnn, Wih0, Whh0,
           bih0, bhh0, Wih1, Whh1, bih1, bhh1, W_fc, b_fc):
    cfg = config_feat.reshape(config_feat.shape[1], config_feat.shape[2])
    sel = jnp.zeros((D_NODE, 1), jnp.float32).at[0, 0].set(1.0)
    w_col = jnp.concatenate([W_gnn.reshape(D_NODE, 1), sel], axis=1)
    bg = b_gnn.reshape(1, 1)
    wih0_t = Wih0.T
    whh0r = Whh0.T.reshape(1, 4)
    b0r = (bih0 + bhh0).reshape(1, 4)
    wih1r = Wih1.T.reshape(1, 4)
    whh1r = Whh1.T.reshape(1, 4)
    b1r = (bih1 + bhh1).reshape(1, 4)
    wfc = W_fc.reshape(1, 2)
    bfc = b_fc.reshape(1, 1)
    return _run(node_feat, cfg, w_col, bg, wih0_t, whh0r, b0r, wih1r, whh1r,
                b1r, wfc, bfc)
